# Initial kernel scaffold; baseline (speedup 1.0000x reference)
import jax
import jax.numpy as jnp
from jax.experimental import pallas as pl


def kernel(idx, targets, token_embedding_table):
    raise NotImplementedError("write your pallas kernel here")


# speedup vs baseline: 1.3680x; 1.3680x over previous
import functools

import jax
import jax.numpy as jnp
from jax import lax
from jax.experimental import pallas as pl
from jax.experimental.pallas import tpu as pltpu
from jax.experimental.pallas import tpu_sc as plsc

V = 1000
B = 1024
T = 50
N = B * T

_LANES = 16
_CHUNK = 32


def _lse_body(tab_ref, out_ref):
    x = tab_ref[...]
    m = jnp.max(x, axis=1)
    out_ref[...] = m + jnp.log(jnp.sum(jnp.exp(x - m[:, None]), axis=1))


def _row_lse(table):
    return pl.pallas_call(
        _lse_body,
        out_shape=jax.ShapeDtypeStruct((V,), jnp.float32),
    )(table)


def _make_sc_kernel():
    info = plsc.get_sparse_core_info()
    nc, ns = info.num_cores, info.num_subcores
    nw = nc * ns
    tok_per_w = N // nw
    n_chunks = tok_per_w // _CHUNK

    mesh = plsc.VectorSubcoreMesh(core_axis_name="c", subcore_axis_name="s")

    @functools.partial(
        pl.kernel,
        mesh=mesh,
        compiler_params=pltpu.CompilerParams(
            needs_layout_passes=False, use_tc_tiling_on_sc=False
        ),
        out_type=[
            jax.ShapeDtypeStruct((N, V), jnp.float32),
            jax.ShapeDtypeStruct((nw, _LANES), jnp.float32),
        ],
        scratch_types=[
            pltpu.VMEM((tok_per_w,), jnp.int32),
            pltpu.VMEM((tok_per_w,), jnp.int32),
            pltpu.VMEM((V,), jnp.float32),
            pltpu.VMEM((_CHUNK, V), jnp.float32),
            pltpu.VMEM((_LANES,), jnp.float32),
            pltpu.SemaphoreType.DMA,
        ],
    )
    def sc_kernel(idx_hbm, tgt_hbm, table_hbm, lse_hbm,
                  out_hbm, part_hbm,
                  idx_v, tgt_v, lse_v, buf, stage_v, sem):
        wid = lax.axis_index("s") * nc + lax.axis_index("c")
        base = wid * tok_per_w
        pltpu.sync_copy(idx_hbm.at[pl.ds(base, tok_per_w)], idx_v)
        pltpu.sync_copy(tgt_hbm.at[pl.ds(base, tok_per_w)], tgt_v)
        pltpu.sync_copy(lse_hbm, lse_v)

        lane = lax.iota(jnp.int32, _LANES)

        def chunk_step(g, acc):
            off = g * _CHUNK
            pltpu.async_copy(
                table_hbm.at[idx_v.at[pl.ds(off, _CHUNK)]], buf, sem
            ).wait()
            for j in range(_CHUNK // _LANES):
                idx_vals = idx_v[pl.ds(off + j * _LANES, _LANES)]
                tgt_vals = tgt_v[pl.ds(off + j * _LANES, _LANES)]
                row_ids = lane + (j * _LANES)
                lse_vals = plsc.load_gather(lse_v, [idx_vals])
                tl = plsc.load_gather(buf, [row_ids, tgt_vals])
                acc = acc + (lse_vals - tl)
            pltpu.sync_copy(buf, out_hbm.at[pl.ds(base + off, _CHUNK)])
            return acc

        acc = lax.fori_loop(0, n_chunks, chunk_step, jnp.zeros((_LANES,), jnp.float32))
        stage_v[...] = acc
        pltpu.sync_copy(stage_v, part_hbm.at[wid])

    return sc_kernel


def kernel(idx, targets, token_embedding_table):
    idx_flat = idx.reshape(N).astype(jnp.int32)
    tgt_flat = targets.reshape(N).astype(jnp.int32)
    table = token_embedding_table.astype(jnp.float32)
    lse = _row_lse(table)
    out_flat, partials = _make_sc_kernel()(idx_flat, tgt_flat, table, lse)
    logits = out_flat.reshape(B, T, V)
    loss = jnp.sum(partials) / N
    return logits, loss


# speedup vs baseline: 1.4317x; 1.0465x over previous
import functools

import jax
import jax.numpy as jnp
from jax import lax
from jax.experimental import pallas as pl
from jax.experimental.pallas import tpu as pltpu
from jax.experimental.pallas import tpu_sc as plsc

V = 1000
B = 1024
T = 50
N = B * T

_LANES = 16
_CHUNK = 32
_NBUF = 2


def _lse_body(tab_ref, out_ref):
    x = tab_ref[...]
    m = jnp.max(x, axis=1)
    out_ref[...] = m + jnp.log(jnp.sum(jnp.exp(x - m[:, None]), axis=1))


def _row_lse(table):
    return pl.pallas_call(
        _lse_body,
        out_shape=jax.ShapeDtypeStruct((V,), jnp.float32),
    )(table)


def _make_sc_kernel():
    info = plsc.get_sparse_core_info()
    nc, ns = info.num_cores, info.num_subcores
    nw = nc * ns
    tok_per_w = N // nw
    n_chunks = tok_per_w // _CHUNK

    mesh = plsc.VectorSubcoreMesh(core_axis_name="c", subcore_axis_name="s")

    @functools.partial(
        pl.kernel,
        mesh=mesh,
        compiler_params=pltpu.CompilerParams(
            needs_layout_passes=False, use_tc_tiling_on_sc=False
        ),
        out_type=[
            jax.ShapeDtypeStruct((N, V), jnp.float32),
            jax.ShapeDtypeStruct((nw, _LANES), jnp.float32),
        ],
        scratch_types=[
            pltpu.VMEM((tok_per_w,), jnp.int32),
            pltpu.VMEM((tok_per_w,), jnp.int32),
            pltpu.VMEM((V,), jnp.float32),
            pltpu.VMEM((_NBUF, _CHUNK, V), jnp.float32),
            pltpu.VMEM((_LANES,), jnp.float32),
            pltpu.SemaphoreType.DMA((_NBUF,)),
            pltpu.SemaphoreType.DMA((_NBUF,)),
        ],
    )
    def sc_kernel(idx_hbm, tgt_hbm, table_hbm, lse_hbm,
                  out_hbm, part_hbm,
                  idx_v, tgt_v, lse_v, buf, stage_v, gsem, osem):
        wid = lax.axis_index("s") * nc + lax.axis_index("c")
        base = wid * tok_per_w
        pltpu.sync_copy(idx_hbm.at[pl.ds(base, tok_per_w)], idx_v)
        pltpu.sync_copy(tgt_hbm.at[pl.ds(base, tok_per_w)], tgt_v)
        pltpu.sync_copy(lse_hbm, lse_v)

        lane = lax.iota(jnp.int32, _LANES)

        def gather_copy(g, b):
            return pltpu.make_async_copy(
                table_hbm.at[idx_v.at[pl.ds(g * _CHUNK, _CHUNK)]],
                buf.at[b], gsem.at[b],
            )

        def out_copy(g, b):
            return pltpu.make_async_copy(
                buf.at[b], out_hbm.at[pl.ds(base + g * _CHUNK, _CHUNK)],
                osem.at[b],
            )

        def loss_accum(g, b, acc):
            off = g * _CHUNK
            for j in range(_CHUNK // _LANES):
                idx_vals = idx_v[pl.ds(off + j * _LANES, _LANES)]
                tgt_vals = tgt_v[pl.ds(off + j * _LANES, _LANES)]
                row_ids = lane + (j * _LANES)
                lse_vals = plsc.load_gather(lse_v, [idx_vals])
                tl = plsc.load_gather(buf.at[b], [row_ids, tgt_vals])
                acc = acc + (lse_vals - tl)
            return acc

        for b in range(_NBUF):
            gather_copy(b, b).start()

        def ring_step(gg, acc):
            for b in range(_NBUF):
                g = gg * _NBUF + b
                gather_copy(g, b).wait()
                out_copy(g, b).start()
                acc = loss_accum(g, b, acc)
                out_copy(g, b).wait()
                gather_copy(g + _NBUF, b).start()
            return acc

        acc = lax.fori_loop(0, n_chunks // _NBUF - 1, ring_step,
                            jnp.zeros((_LANES,), jnp.float32))
        for b in range(_NBUF):
            g = n_chunks - _NBUF + b
            gather_copy(g, b).wait()
            out_copy(g, b).start()
            acc = loss_accum(g, b, acc)
            out_copy(g, b).wait()

        stage_v[...] = acc
        pltpu.sync_copy(stage_v, part_hbm.at[wid])

    return sc_kernel


def kernel(idx, targets, token_embedding_table):
    idx_flat = idx.reshape(N).astype(jnp.int32)
    tgt_flat = targets.reshape(N).astype(jnp.int32)
    table = token_embedding_table.astype(jnp.float32)
    lse = _row_lse(table)
    out_flat, partials = _make_sc_kernel()(idx_flat, tgt_flat, table, lse)
    logits = out_flat.reshape(B, T, V)
    loss = jnp.sum(partials) / N
    return logits, loss
